# baseline (speedup 1.0000x reference)
import functools

import jax
import jax.numpy as jnp
from jax import lax
from jax.experimental import pallas as pl
from jax.experimental.pallas import tpu as pltpu
from jax.experimental.pallas import tpu_sc as plsc

_EMB_DIM = 256
_NUM_EMB = 8192
_COMMIT = 0.25
_ROWS = 8192
_BLK = 512
_NBLK = _ROWS // _BLK
_STRIPE = 2816
_NC, _NS = 2, 16
_NW = _NC * _NS
_BPW = _ROWS // _NW


def _assign_body(x_ref, e_ref, eb_ref, idx_ref, loss_ref, e2_ref, acc_ref):
    i = pl.program_id(0)

    @pl.when(i == 0)
    def _init():
        e = e_ref[...]
        e2_ref[...] = jnp.sum(e * e, axis=0, keepdims=True)
        acc_ref[...] = jnp.zeros((1, 1), jnp.float32)

    x = x_ref[...]
    x2 = jnp.sum(x * x, axis=1, keepdims=True)
    mm = jnp.dot(x.astype(jnp.bfloat16), eb_ref[...],
                 preferred_element_type=jnp.float32)
    dist = (x2 - 2.0 * mm) + e2_ref[...]
    bounds = (0, _STRIPE, 2 * _STRIPE, _NUM_EMB)
    mind = acc_v = acc_i = None
    for c in range(3):
        lo, hi = bounds[c], bounds[c + 1]
        dc = dist[:, lo:hi]
        mv = jnp.min(dc, axis=1, keepdims=True)
        mi = (jnp.argmin(dc, axis=1).astype(jnp.int32) + lo).reshape(_BLK, 1)
        if c == 0:
            mind, acc_v, acc_i = mv, mv, mi
        else:
            mind = jnp.minimum(mind, mv)
            take = mv < acc_v
            acc_i = jnp.where(take, mi, acc_i)
            acc_v = jnp.where(take, mv, acc_v)
        acc_v = acc_v.astype(jnp.bfloat16).astype(jnp.float32)
    idx_ref[...] = acc_i.reshape(1, 1, _BLK)
    acc_ref[...] += jnp.sum(mind).reshape(1, 1)

    @pl.when(i == _NBLK - 1)
    def _fin():
        loss_ref[...] = acc_ref[...] * (_COMMIT / (_ROWS * _EMB_DIM))


def _tc_assign(x, embed, embed_bf16, interpret=False):
    return pl.pallas_call(
        _assign_body,
        grid=(_NBLK,),
        in_specs=[pl.BlockSpec((_BLK, _EMB_DIM), lambda i: (i, 0)),
                  pl.BlockSpec((_EMB_DIM, _NUM_EMB), lambda i: (0, 0)),
                  pl.BlockSpec((_EMB_DIM, _NUM_EMB), lambda i: (0, 0))],
        out_specs=[pl.BlockSpec((1, 1, _BLK), lambda i: (i, 0, 0)),
                   pl.BlockSpec((1, 1), lambda i: (0, 0))],
        out_shape=[jax.ShapeDtypeStruct((_NBLK, 1, _BLK), jnp.int32),
                   jax.ShapeDtypeStruct((1, 1), jnp.float32)],
        scratch_shapes=[pltpu.VMEM((1, _NUM_EMB), jnp.float32),
                        pltpu.VMEM((1, 1), jnp.float32)],
        interpret=interpret,
    )(x, embed, embed_bf16)


def _perp_body(c2_ref, perp_ref):
    c = c2_ref[0:1, :] + c2_ref[1:2, :]
    p = c * (1.0 / _ROWS)
    ent = jnp.sum(p * jnp.log(p + 1e-10))
    perp_ref[...] = jnp.exp(-ent).reshape(1, 1)


def _tc_perp(counts2):
    return pl.pallas_call(
        _perp_body,
        out_shape=jax.ShapeDtypeStruct((1, 1), jnp.float32),
    )(counts2)


@functools.cache
def _sc_gather_kernel():
    @functools.partial(
        pl.kernel,
        out_type=[jax.ShapeDtypeStruct((_ROWS, _EMB_DIM), jnp.float32),
                  jax.ShapeDtypeStruct((_NC, _NUM_EMB), jnp.float32)],
        mesh=plsc.VectorSubcoreMesh(core_axis_name="c", subcore_axis_name="s"),
        scratch_types=[pltpu.VMEM((_BPW, _EMB_DIM), jnp.float32),
                       pltpu.VMEM((2, 128), jnp.int32),
                       pltpu.VMEM((128,), jnp.float32),
                       pltpu.VMEM_SHARED((_NUM_EMB,), jnp.float32),
                       pltpu.SemaphoreType.DMA],
    )
    def gather(table_hbm, idx2_hbm, ones_hbm, zeros_hbm,
               q_hbm, cnt_hbm, rows_v, idx2_v, ones_v, cnt_sh, sem):
        c = lax.axis_index("c")
        s = lax.axis_index("s")
        wid = s * _NC + c
        base = wid * _BPW

        @pl.when(s == 0)
        def _zero():
            pltpu.sync_copy(zeros_hbm, cnt_sh)

        plsc.subcore_barrier()
        pltpu.sync_copy(idx2_hbm.at[pl.ds(wid * 2, 2)], idx2_v)
        for j in range(2):
            pltpu.async_copy(table_hbm.at[idx2_v.at[j]],
                             rows_v.at[pl.ds(j * 128, 128)], sem).wait()
        pltpu.sync_copy(rows_v, q_hbm.at[pl.ds(base, _BPW)])
        pltpu.sync_copy(ones_hbm, ones_v)
        for j in range(2):
            pltpu.sync_copy(ones_v, cnt_sh.at[idx2_v.at[j]], add=True)
        plsc.subcore_barrier()

        @pl.when(s == 0)
        def _out():
            pltpu.sync_copy(cnt_sh, cnt_hbm.at[c])

    return gather


def kernel(inputs, embed):
    x = jnp.transpose(inputs, (0, 2, 3, 1)).reshape(_ROWS, _EMB_DIM)
    idx3, loss = _tc_assign(x, embed, embed.astype(jnp.bfloat16))
    idx = idx3.reshape(_ROWS)
    q, counts2 = _sc_gather_kernel()(
        embed.T, idx.reshape(_ROWS // 128, 128),
        jnp.ones((128,), jnp.float32), jnp.zeros((_NUM_EMB,), jnp.float32))
    perp = _tc_perp(counts2)
    qt = jnp.transpose(q.reshape(8, 32, 32, _EMB_DIM), (0, 3, 1, 2))
    return qt, loss.reshape(()), perp.reshape(())


# speedup vs baseline: 1.0213x; 1.0213x over previous
import functools

import jax
import jax.numpy as jnp
from jax import lax
from jax.experimental import pallas as pl
from jax.experimental.pallas import tpu as pltpu
from jax.experimental.pallas import tpu_sc as plsc

_EMB_DIM = 256
_NUM_EMB = 8192
_COMMIT = 0.25
_ROWS = 8192
_BLK = 512
_NBLK = _ROWS // _BLK
_STRIPE = 2816
_NC, _NS = 2, 16
_NW = _NC * _NS
_BPW = _ROWS // _NW


def _assign_body(x_ref, e_ref, eb_ref, idx_ref, loss_ref, e2_ref, acc_ref):
    i = pl.program_id(0)

    @pl.when(i == 0)
    def _init():
        e = e_ref[...]
        e2_ref[...] = jnp.sum(e * e, axis=0, keepdims=True)
        acc_ref[...] = jnp.zeros((1, 1), jnp.float32)

    x = x_ref[...]
    x2 = jnp.sum(x * x, axis=1, keepdims=True)
    mm = jnp.dot(x.astype(jnp.bfloat16), eb_ref[...],
                 preferred_element_type=jnp.float32)
    dist = (x2 - 2.0 * mm) + e2_ref[...]
    bounds = (0, _STRIPE, 2 * _STRIPE, _NUM_EMB)
    mind = acc_v = acc_i = None
    for c in range(3):
        lo, hi = bounds[c], bounds[c + 1]
        dc = dist[:, lo:hi]
        mv = jnp.min(dc, axis=1, keepdims=True)
        mi = (jnp.argmin(dc, axis=1).astype(jnp.int32) + lo).reshape(_BLK, 1)
        if c == 0:
            mind, acc_v, acc_i = mv, mv, mi
        else:
            mind = jnp.minimum(mind, mv)
            take = mv < acc_v
            acc_i = jnp.where(take, mi, acc_i)
            acc_v = jnp.where(take, mv, acc_v)
        acc_v = acc_v.astype(jnp.bfloat16).astype(jnp.float32)
    idx_ref[...] = acc_i.reshape(1, 1, _BLK)
    acc_ref[...] += jnp.sum(mind).reshape(1, 1)

    @pl.when(i == _NBLK - 1)
    def _fin():
        loss_ref[...] = acc_ref[...] * (_COMMIT / (_ROWS * _EMB_DIM))


def _tc_assign(x, embed, embed_bf16, interpret=False):
    return pl.pallas_call(
        _assign_body,
        grid=(_NBLK,),
        in_specs=[pl.BlockSpec((_BLK, _EMB_DIM), lambda i: (i, 0)),
                  pl.BlockSpec((_EMB_DIM, _NUM_EMB), lambda i: (0, 0)),
                  pl.BlockSpec((_EMB_DIM, _NUM_EMB), lambda i: (0, 0))],
        out_specs=[pl.BlockSpec((1, 1, _BLK), lambda i: (i, 0, 0)),
                   pl.BlockSpec((1, 1), lambda i: (0, 0))],
        out_shape=[jax.ShapeDtypeStruct((_NBLK, 1, _BLK), jnp.int32),
                   jax.ShapeDtypeStruct((1, 1), jnp.float32)],
        scratch_shapes=[pltpu.VMEM((1, _NUM_EMB), jnp.float32),
                        pltpu.VMEM((1, 1), jnp.float32)],
        interpret=interpret,
    )(x, embed, embed_bf16)


def _perp_body(c2_ref, perp_ref):
    c = c2_ref[0:1, :] + c2_ref[1:2, :]
    p = c * (1.0 / _ROWS)
    ent = jnp.sum(p * jnp.log(p + 1e-10))
    perp_ref[...] = jnp.exp(-ent).reshape(1, 1)


def _tc_perp(counts2):
    return pl.pallas_call(
        _perp_body,
        out_shape=jax.ShapeDtypeStruct((1, 1), jnp.float32),
    )(counts2)


@functools.cache
def _sc_gather_kernel():
    @functools.partial(
        pl.kernel,
        out_type=[jax.ShapeDtypeStruct((_ROWS, _EMB_DIM), jnp.float32),
                  jax.ShapeDtypeStruct((_NC, _NUM_EMB), jnp.float32)],
        mesh=plsc.VectorSubcoreMesh(core_axis_name="c", subcore_axis_name="s"),
        scratch_types=[pltpu.VMEM((_BPW, _EMB_DIM), jnp.float32),
                       pltpu.VMEM((2, 128), jnp.int32),
                       pltpu.VMEM((128,), jnp.float32),
                       pltpu.VMEM_SHARED((_NUM_EMB,), jnp.float32),
                       pltpu.SemaphoreType.DMA],
    )
    def gather(table_hbm, idx2_hbm, ones_hbm, zeros_hbm,
               q_hbm, cnt_hbm, rows_v, idx2_v, ones_v, cnt_sh, sem):
        c = lax.axis_index("c")
        s = lax.axis_index("s")
        wid = s * _NC + c
        base = wid * _BPW

        @pl.when(s == 0)
        def _zero():
            pltpu.sync_copy(zeros_hbm, cnt_sh)

        plsc.subcore_barrier()
        pltpu.sync_copy(idx2_hbm.at[pl.ds(wid * 2, 2)], idx2_v)
        cp0 = pltpu.async_copy(table_hbm.at[idx2_v.at[0]],
                               rows_v.at[pl.ds(0, 128)], sem)
        cp1 = pltpu.async_copy(table_hbm.at[idx2_v.at[1]],
                               rows_v.at[pl.ds(128, 128)], sem)
        pltpu.sync_copy(ones_hbm, ones_v)
        for j in range(2):
            pltpu.sync_copy(ones_v, cnt_sh.at[idx2_v.at[j]], add=True)
        cp0.wait()
        cp1.wait()
        pltpu.sync_copy(rows_v, q_hbm.at[pl.ds(base, _BPW)])
        plsc.subcore_barrier()

        @pl.when(s == 0)
        def _out():
            pltpu.sync_copy(cnt_sh, cnt_hbm.at[c])

    return gather


def kernel(inputs, embed):
    x = jnp.transpose(inputs, (0, 2, 3, 1)).reshape(_ROWS, _EMB_DIM)
    idx3, loss = _tc_assign(x, embed, embed.astype(jnp.bfloat16))
    idx = idx3.reshape(_ROWS)
    q, counts2 = _sc_gather_kernel()(
        embed.T, idx.reshape(_ROWS // 128, 128),
        jnp.ones((128,), jnp.float32), jnp.zeros((_NUM_EMB,), jnp.float32))
    perp = _tc_perp(counts2)
    qt = jnp.transpose(q.reshape(8, 32, 32, _EMB_DIM), (0, 3, 1, 2))
    return qt, loss.reshape(()), perp.reshape(())
